# baseline (speedup 1.0000x reference)
import functools

import jax
import jax.numpy as jnp
from jax import lax
from jax.experimental import pallas as pl
from jax.experimental.pallas import tpu as pltpu
from jax.experimental.pallas import tpu_sc as plsc

_B = 4096
_D = 256
_K = 8192

_NC, _NS = 2, 16
_NW = _NC * _NS
_CHUNK = _B // _NW

_BB = 1024
_KB = 512


_NPB = _K // _KB
_NAB = _B // _BB


def _pa_body(emb_ref, w_ref, b_ref, x1_ref, x2_ref,
             qcb_ref, idx1_ref, idx2_ref, qcb2_scr, cbsq_scr):
    i = pl.program_id(0)

    @pl.when(i < _NPB)
    def _proj_phase():
        mm = lax.dot_general(emb_ref[...], w_ref[...],
                             (((1,), (1,)), ((), ())),
                             preferred_element_type=jnp.float32)
        qcb = mm + b_ref[...]
        qcb_ref[...] = qcb
        qcb2_scr[pl.ds(i * _KB, _KB), :] = qcb + qcb
        cbsq_scr[:, pl.ds(i * _KB, _KB)] = (
            jnp.sum(qcb * qcb, axis=1).reshape(1, _KB))

    @pl.when(i >= _NPB)
    def _argmin_phase():
        qcb2 = qcb2_scr[...]
        cb_sq = cbsq_scr[...]
        for x_ref, idx_ref in ((x1_ref, idx1_ref), (x2_ref, idx2_ref)):
            x = x_ref[...]
            x_sq = jnp.sum(x * x, axis=1, keepdims=True)
            mm2 = lax.dot_general(x, qcb2, (((1,), (1,)), ((), ())),
                                  preferred_element_type=jnp.float32)
            dist = x_sq + cb_sq - mm2
            idx_ref[...] = jnp.argmin(dist, axis=1).astype(jnp.int32)


def _proj_argmin(emb, w, b2d, x1, x2):
    return pl.pallas_call(
        _pa_body,
        grid=(_NPB + _NAB,),
        in_specs=[
            pl.BlockSpec((_KB, _D), lambda i: (jnp.minimum(i, _NPB - 1), 0)),
            pl.BlockSpec((_D, _D), lambda i: (0, 0)),
            pl.BlockSpec((1, _D), lambda i: (0, 0)),
            pl.BlockSpec((_BB, _D), lambda i: (jnp.maximum(i - _NPB, 0), 0)),
            pl.BlockSpec((_BB, _D), lambda i: (jnp.maximum(i - _NPB, 0), 0)),
        ],
        out_specs=[
            pl.BlockSpec((_KB, _D), lambda i: (jnp.minimum(i, _NPB - 1), 0)),
            pl.BlockSpec((_BB,), lambda i: (jnp.maximum(i - _NPB, 0),)),
            pl.BlockSpec((_BB,), lambda i: (jnp.maximum(i - _NPB, 0),)),
        ],
        out_shape=[
            jax.ShapeDtypeStruct((_K, _D), jnp.float32),
            jax.ShapeDtypeStruct((_B,), jnp.int32),
            jax.ShapeDtypeStruct((_B,), jnp.int32),
        ],
        scratch_shapes=[
            pltpu.VMEM((_K, _D), jnp.float32),
            pltpu.VMEM((1, _K), jnp.float32),
        ],
    )(emb, w, b2d, x1, x2)


@functools.partial(
    pl.kernel,
    out_type=(
        jax.ShapeDtypeStruct((_B, _D), jnp.float32),
        jax.ShapeDtypeStruct((_B, _D), jnp.float32),
        jax.ShapeDtypeStruct((_NW, _K), jnp.float32),
        jax.ShapeDtypeStruct((_NW, _K), jnp.float32),
    ),
    mesh=plsc.VectorSubcoreMesh(core_axis_name="c", subcore_axis_name="s"),
    compiler_params=pltpu.CompilerParams(needs_layout_passes=False),
    scratch_types=[
        pltpu.VMEM((_CHUNK,), jnp.int32),
        pltpu.VMEM((_CHUNK,), jnp.int32),
        pltpu.VMEM((_CHUNK, _D), jnp.float32),
        pltpu.VMEM((_CHUNK, _D), jnp.float32),
        pltpu.VMEM((_K,), jnp.float32),
        pltpu.VMEM((_K,), jnp.float32),
        pltpu.SemaphoreType.DMA,
        pltpu.SemaphoreType.DMA,
    ],
)
def _sc_gather_count(qcb_hbm, idx1_hbm, idx2_hbm,
                     out1_hbm, out2_hbm, cnt1_hbm, cnt2_hbm,
                     idx1_v, idx2_v, rows1_v, rows2_v, cnt1_v, cnt2_v,
                     sem1, sem2):
    wid = lax.axis_index("s") * _NC + lax.axis_index("c")
    base = wid * _CHUNK
    ones = jnp.ones((16,), jnp.float32)
    zeros = jnp.zeros((16,), jnp.float32)

    pltpu.sync_copy(idx1_hbm.at[pl.ds(base, _CHUNK)], idx1_v)
    pltpu.sync_copy(idx2_hbm.at[pl.ds(base, _CHUNK)], idx2_v)
    g1 = pltpu.async_copy(qcb_hbm.at[idx1_v], rows1_v, sem1)
    g2 = pltpu.async_copy(qcb_hbm.at[idx2_v], rows2_v, sem2)

    def _zero(i, carry):
        for u in range(8):
            cnt1_v[pl.ds((i * 8 + u) * 16, 16)] = zeros
            cnt2_v[pl.ds((i * 8 + u) * 16, 16)] = zeros
        return carry
    lax.fori_loop(0, _K // 128, _zero, 0)

    for j in range(_CHUNK // 16):
        plsc.addupdate_scatter(cnt1_v, [idx1_v[pl.ds(j * 16, 16)]], ones)
        plsc.addupdate_scatter(cnt2_v, [idx2_v[pl.ds(j * 16, 16)]], ones)
    pltpu.sync_copy(cnt1_v, cnt1_hbm.at[wid])
    pltpu.sync_copy(cnt2_v, cnt2_hbm.at[wid])

    g1.wait()
    pltpu.sync_copy(rows1_v, out1_hbm.at[pl.ds(base, _CHUNK)])
    g2.wait()
    pltpu.sync_copy(rows2_v, out2_hbm.at[pl.ds(base, _CHUNK)])


def _final_body(x1_ref, x2_ref, q1_ref, q2_ref, cnt1_ref, cnt2_ref,
                out_ref, acc_ref):
    i = pl.program_id(0)
    nb = pl.num_programs(0)

    @pl.when(i == 0)
    def _init():
        for t in range(4):
            acc_ref[t] = jnp.float32(0.0)

    x1 = x1_ref[...]
    x2 = x2_ref[...]
    q1 = q1_ref[...]
    q2 = q2_ref[...]
    d1 = x1 - q1
    d2 = x2 - q2
    d3 = q2 - x1
    d4 = q1 - x2
    acc_ref[0] = acc_ref[0] + jnp.sum(d1 * d1)
    acc_ref[1] = acc_ref[1] + jnp.sum(d2 * d2)
    acc_ref[2] = acc_ref[2] + jnp.sum(d3 * d3)
    acc_ref[3] = acc_ref[3] + jnp.sum(d4 * d4)

    @pl.when(i == nb - 1)
    def _done():
        n = jnp.float32(_B * _D)
        m1 = acc_ref[0] / n
        m2 = acc_ref[1] / n
        m3 = acc_ref[2] / n
        m4 = acc_ref[3] / n
        scr_loss = jnp.float32(0.5) * m1
        fwd = m2 + m1 + jnp.float32(0.5) * m3 + jnp.float32(0.5) * m4
        ribo_loss = jnp.float32(0.5) * m2 + jnp.float32(0.25) * fwd

        c1 = jnp.sum(cnt1_ref[...], axis=0, keepdims=True)
        c2 = jnp.sum(cnt2_ref[...], axis=0, keepdims=True)
        binv = jnp.float32(1.0 / _B)
        p1 = c1 * binv
        p2 = c2 * binv
        eps = jnp.float32(1e-10)
        perp1 = jnp.exp(-jnp.sum(p1 * jnp.log(p1 + eps)))
        perp2 = jnp.exp(-jnp.sum(p2 * jnp.log(p2 + eps)))

        r_mask = c1 > 0.0
        b_mask = c2 > 0.0
        union = jnp.sum((r_mask | b_mask).astype(jnp.float32))
        inter = jnp.sum((r_mask & b_mask).astype(jnp.float32))
        r_cnt = jnp.sum(r_mask.astype(jnp.float32))
        b_cnt = jnp.sum(b_mask.astype(jnp.float32))
        kinv = jnp.float32(1.0 / _K)

        out_ref[0] = scr_loss
        out_ref[1] = ribo_loss
        out_ref[2] = perp1
        out_ref[3] = perp2
        out_ref[4] = union * kinv
        out_ref[5] = r_cnt * kinv
        out_ref[6] = b_cnt * kinv
        out_ref[7] = inter * kinv
        out_ref[8] = union * kinv
        out_ref[9] = union


_FB = 1024


def _finalize(x1, x2, q1, q2, cnt1, cnt2):
    nb = _B // _FB
    return pl.pallas_call(
        _final_body,
        grid=(nb,),
        in_specs=[
            pl.BlockSpec((_FB, _D), lambda i: (i, 0)),
            pl.BlockSpec((_FB, _D), lambda i: (i, 0)),
            pl.BlockSpec((_FB, _D), lambda i: (i, 0)),
            pl.BlockSpec((_FB, _D), lambda i: (i, 0)),
            pl.BlockSpec((_NW, _K), lambda i: (0, 0)),
            pl.BlockSpec((_NW, _K), lambda i: (0, 0)),
        ],
        out_specs=pl.BlockSpec(memory_space=pltpu.SMEM),
        out_shape=jax.ShapeDtypeStruct((16,), jnp.float32),
        scratch_shapes=[pltpu.SMEM((4,), jnp.float32)],
    )(x1, x2, q1, q2, cnt1, cnt2)


def kernel(scRNA_semantic, ribo_semantic, flag, embedding_weight, proj_W, proj_b):
    del flag
    qcb, idx1, idx2 = _proj_argmin(embedding_weight, proj_W,
                                   proj_b.reshape(1, _D),
                                   scRNA_semantic, ribo_semantic)
    q1, q2, cnt1, cnt2 = _sc_gather_count(qcb, idx1, idx2)
    s = _finalize(scRNA_semantic, ribo_semantic, q1, q2, cnt1, cnt2)
    return (q1, q2, s[0], s[1], s[2], s[3], s[4:10])


# speedup vs baseline: 1.0007x; 1.0007x over previous
import functools

import jax
import jax.numpy as jnp
from jax import lax
from jax.experimental import pallas as pl
from jax.experimental.pallas import tpu as pltpu
from jax.experimental.pallas import tpu_sc as plsc

_B = 4096
_D = 256
_K = 8192

_NC, _NS = 2, 16
_NW = _NC * _NS
_CHUNK = _B // _NW

_BB = 512
_KB = 512


_NPB = _K // _KB
_NAB = _B // _BB


def _pa_body(emb_ref, w_ref, b_ref, x1_ref, x2_ref,
             qcb_ref, idx1_ref, idx2_ref, qcb2_scr, cbsq_scr):
    i = pl.program_id(0)

    @pl.when(i < _NPB)
    def _proj_phase():
        mm = lax.dot_general(emb_ref[...], w_ref[...],
                             (((1,), (1,)), ((), ())),
                             preferred_element_type=jnp.float32)
        qcb = mm + b_ref[...]
        qcb_ref[...] = qcb
        qcb2_scr[pl.ds(i * _KB, _KB), :] = qcb + qcb
        cbsq_scr[:, pl.ds(i * _KB, _KB)] = (
            jnp.sum(qcb * qcb, axis=1).reshape(1, _KB))

    @pl.when(i >= _NPB)
    def _argmin_phase():
        qcb2 = qcb2_scr[...]
        cb_sq = cbsq_scr[...]
        for x_ref, idx_ref in ((x1_ref, idx1_ref), (x2_ref, idx2_ref)):
            x = x_ref[...]
            x_sq = jnp.sum(x * x, axis=1, keepdims=True)
            mm2 = lax.dot_general(x, qcb2, (((1,), (1,)), ((), ())),
                                  preferred_element_type=jnp.float32)
            dist = x_sq + cb_sq - mm2
            idx_ref[...] = jnp.argmin(dist, axis=1).astype(jnp.int32)


def _proj_argmin(emb, w, b2d, x1, x2):
    return pl.pallas_call(
        _pa_body,
        grid=(_NPB + _NAB,),
        in_specs=[
            pl.BlockSpec((_KB, _D), lambda i: (jnp.minimum(i, _NPB - 1), 0)),
            pl.BlockSpec((_D, _D), lambda i: (0, 0)),
            pl.BlockSpec((1, _D), lambda i: (0, 0)),
            pl.BlockSpec((_BB, _D), lambda i: (jnp.maximum(i - _NPB, 0), 0)),
            pl.BlockSpec((_BB, _D), lambda i: (jnp.maximum(i - _NPB, 0), 0)),
        ],
        out_specs=[
            pl.BlockSpec((_KB, _D), lambda i: (jnp.minimum(i, _NPB - 1), 0)),
            pl.BlockSpec((_BB,), lambda i: (jnp.maximum(i - _NPB, 0),)),
            pl.BlockSpec((_BB,), lambda i: (jnp.maximum(i - _NPB, 0),)),
        ],
        out_shape=[
            jax.ShapeDtypeStruct((_K, _D), jnp.float32),
            jax.ShapeDtypeStruct((_B,), jnp.int32),
            jax.ShapeDtypeStruct((_B,), jnp.int32),
        ],
        scratch_shapes=[
            pltpu.VMEM((_K, _D), jnp.float32),
            pltpu.VMEM((1, _K), jnp.float32),
        ],
    )(emb, w, b2d, x1, x2)


@functools.partial(
    pl.kernel,
    out_type=(
        jax.ShapeDtypeStruct((_B, _D), jnp.float32),
        jax.ShapeDtypeStruct((_B, _D), jnp.float32),
        jax.ShapeDtypeStruct((_NW, _K), jnp.float32),
        jax.ShapeDtypeStruct((_NW, _K), jnp.float32),
    ),
    mesh=plsc.VectorSubcoreMesh(core_axis_name="c", subcore_axis_name="s"),
    compiler_params=pltpu.CompilerParams(needs_layout_passes=False),
    scratch_types=[
        pltpu.VMEM((_CHUNK,), jnp.int32),
        pltpu.VMEM((_CHUNK,), jnp.int32),
        pltpu.VMEM((_CHUNK, _D), jnp.float32),
        pltpu.VMEM((_CHUNK, _D), jnp.float32),
        pltpu.VMEM((_K,), jnp.float32),
        pltpu.VMEM((_K,), jnp.float32),
        pltpu.SemaphoreType.DMA,
        pltpu.SemaphoreType.DMA,
    ],
)
def _sc_gather_count(qcb_hbm, idx1_hbm, idx2_hbm,
                     out1_hbm, out2_hbm, cnt1_hbm, cnt2_hbm,
                     idx1_v, idx2_v, rows1_v, rows2_v, cnt1_v, cnt2_v,
                     sem1, sem2):
    wid = lax.axis_index("s") * _NC + lax.axis_index("c")
    base = wid * _CHUNK
    ones = jnp.ones((16,), jnp.float32)
    zeros = jnp.zeros((16,), jnp.float32)

    pltpu.sync_copy(idx1_hbm.at[pl.ds(base, _CHUNK)], idx1_v)
    pltpu.sync_copy(idx2_hbm.at[pl.ds(base, _CHUNK)], idx2_v)
    g1 = pltpu.async_copy(qcb_hbm.at[idx1_v], rows1_v, sem1)
    g2 = pltpu.async_copy(qcb_hbm.at[idx2_v], rows2_v, sem2)

    def _zero(i, carry):
        for u in range(8):
            cnt1_v[pl.ds((i * 8 + u) * 16, 16)] = zeros
            cnt2_v[pl.ds((i * 8 + u) * 16, 16)] = zeros
        return carry
    lax.fori_loop(0, _K // 128, _zero, 0)

    for j in range(_CHUNK // 16):
        plsc.addupdate_scatter(cnt1_v, [idx1_v[pl.ds(j * 16, 16)]], ones)
        plsc.addupdate_scatter(cnt2_v, [idx2_v[pl.ds(j * 16, 16)]], ones)
    pltpu.sync_copy(cnt1_v, cnt1_hbm.at[wid])
    pltpu.sync_copy(cnt2_v, cnt2_hbm.at[wid])

    g1.wait()
    pltpu.sync_copy(rows1_v, out1_hbm.at[pl.ds(base, _CHUNK)])
    g2.wait()
    pltpu.sync_copy(rows2_v, out2_hbm.at[pl.ds(base, _CHUNK)])


def _final_body(x1_ref, x2_ref, q1_ref, q2_ref, cnt1_ref, cnt2_ref,
                out_ref, acc_ref):
    i = pl.program_id(0)
    nb = pl.num_programs(0)

    @pl.when(i == 0)
    def _init():
        for t in range(4):
            acc_ref[t] = jnp.float32(0.0)

    x1 = x1_ref[...]
    x2 = x2_ref[...]
    q1 = q1_ref[...]
    q2 = q2_ref[...]
    d1 = x1 - q1
    d2 = x2 - q2
    d3 = q2 - x1
    d4 = q1 - x2
    acc_ref[0] = acc_ref[0] + jnp.sum(d1 * d1)
    acc_ref[1] = acc_ref[1] + jnp.sum(d2 * d2)
    acc_ref[2] = acc_ref[2] + jnp.sum(d3 * d3)
    acc_ref[3] = acc_ref[3] + jnp.sum(d4 * d4)

    @pl.when(i == nb - 1)
    def _done():
        n = jnp.float32(_B * _D)
        m1 = acc_ref[0] / n
        m2 = acc_ref[1] / n
        m3 = acc_ref[2] / n
        m4 = acc_ref[3] / n
        scr_loss = jnp.float32(0.5) * m1
        fwd = m2 + m1 + jnp.float32(0.5) * m3 + jnp.float32(0.5) * m4
        ribo_loss = jnp.float32(0.5) * m2 + jnp.float32(0.25) * fwd

        c1 = jnp.sum(cnt1_ref[...], axis=0, keepdims=True)
        c2 = jnp.sum(cnt2_ref[...], axis=0, keepdims=True)
        binv = jnp.float32(1.0 / _B)
        p1 = c1 * binv
        p2 = c2 * binv
        eps = jnp.float32(1e-10)
        perp1 = jnp.exp(-jnp.sum(p1 * jnp.log(p1 + eps)))
        perp2 = jnp.exp(-jnp.sum(p2 * jnp.log(p2 + eps)))

        r_mask = c1 > 0.0
        b_mask = c2 > 0.0
        union = jnp.sum((r_mask | b_mask).astype(jnp.float32))
        inter = jnp.sum((r_mask & b_mask).astype(jnp.float32))
        r_cnt = jnp.sum(r_mask.astype(jnp.float32))
        b_cnt = jnp.sum(b_mask.astype(jnp.float32))
        kinv = jnp.float32(1.0 / _K)

        out_ref[0] = scr_loss
        out_ref[1] = ribo_loss
        out_ref[2] = perp1
        out_ref[3] = perp2
        out_ref[4] = union * kinv
        out_ref[5] = r_cnt * kinv
        out_ref[6] = b_cnt * kinv
        out_ref[7] = inter * kinv
        out_ref[8] = union * kinv
        out_ref[9] = union


_FB = 1024


def _finalize(x1, x2, q1, q2, cnt1, cnt2):
    nb = _B // _FB
    return pl.pallas_call(
        _final_body,
        grid=(nb,),
        in_specs=[
            pl.BlockSpec((_FB, _D), lambda i: (i, 0)),
            pl.BlockSpec((_FB, _D), lambda i: (i, 0)),
            pl.BlockSpec((_FB, _D), lambda i: (i, 0)),
            pl.BlockSpec((_FB, _D), lambda i: (i, 0)),
            pl.BlockSpec((_NW, _K), lambda i: (0, 0)),
            pl.BlockSpec((_NW, _K), lambda i: (0, 0)),
        ],
        out_specs=pl.BlockSpec(memory_space=pltpu.SMEM),
        out_shape=jax.ShapeDtypeStruct((16,), jnp.float32),
        scratch_shapes=[pltpu.SMEM((4,), jnp.float32)],
    )(x1, x2, q1, q2, cnt1, cnt2)


def kernel(scRNA_semantic, ribo_semantic, flag, embedding_weight, proj_W, proj_b):
    del flag
    qcb, idx1, idx2 = _proj_argmin(embedding_weight, proj_W,
                                   proj_b.reshape(1, _D),
                                   scRNA_semantic, ribo_semantic)
    q1, q2, cnt1, cnt2 = _sc_gather_count(qcb, idx1, idx2)
    s = _finalize(scRNA_semantic, ribo_semantic, q1, q2, cnt1, cnt2)
    return (q1, q2, s[0], s[1], s[2], s[3], s[4:10])
